# baseline (speedup 1.0000x reference)
import functools

import jax
import jax.numpy as jnp
from jax import lax
from jax.experimental import pallas as pl
from jax.experimental.pallas import tpu as pltpu
from jax.experimental.pallas import tpu_sc as plsc

EMBED_DIM = 64
SCALE = 8.0
NUM_WORKERS = 32
BW = 128
NBUF = 4
LOOKAHEAD = 2
LANES = 16


def _emb_kernel(n_seq, n_batch):
    assert n_batch == NUM_WORKERS * BW
    assert n_seq % NBUF == 0 and n_seq >= 2 * NBUF
    mesh = plsc.VectorSubcoreMesh(core_axis_name="c", subcore_axis_name="s")

    @functools.partial(
        pl.kernel,
        mesh=mesh,
        out_type=jax.ShapeDtypeStruct((n_seq, EMBED_DIM, n_batch), jnp.float32),
        scratch_types=[
            pltpu.VMEM((NBUF, BW), jnp.int32),
            pltpu.VMEM((NBUF, BW), jnp.int32),
            pltpu.VMEM((NBUF, BW, 128), jnp.float32),
            pltpu.VMEM((NBUF, EMBED_DIM, BW), jnp.float32),
            pltpu.SemaphoreType.DMA((NBUF,)),
            pltpu.SemaphoreType.DMA((NBUF,)),
        ],
        compiler_params=pltpu.CompilerParams(needs_layout_passes=False),
    )
    def emb(xt_hbm, pairs_hbm, out_hbm, idxr, idxh, gbuf, sbuf, sem_g, sem_o):
        wid = lax.axis_index("s") * 2 + lax.axis_index("c")
        b0 = wid * BW

        lane = lax.iota(jnp.int32, LANES)
        row_ids = [lane + (l0 * LANES) for l0 in range(BW // LANES)]

        def fire_gather(g, sp):
            pltpu.sync_copy(xt_hbm.at[g, pl.ds(b0, BW)], idxr.at[sp])
            for k in range(BW // LANES):
                sl = pl.ds(k * LANES, LANES)
                idxh[sp, sl] = lax.shift_right_logical(idxr[sp, sl], 1)
            pltpu.async_copy(
                pairs_hbm.at[idxh.at[sp]], gbuf.at[sp], sem_g.at[sp]
            )

        def wait_gather(sp):
            pltpu.make_async_copy(
                pairs_hbm.at[idxh.at[sp]], gbuf.at[sp], sem_g.at[sp]
            ).wait()

        def store_chunk(g, sp):
            pltpu.async_copy(
                sbuf.at[sp], out_hbm.at[g, :, pl.ds(b0, BW)], sem_o.at[sp]
            )

        def wait_store(g, sp):
            pltpu.make_async_copy(
                sbuf.at[sp], out_hbm.at[g, :, pl.ds(b0, BW)], sem_o.at[sp]
            ).wait()

        def compact(sp):
            half = []
            for l0 in range(BW // LANES):
                iv = idxr[sp, pl.ds(l0 * LANES, LANES)]
                half.append((iv & 1) * EMBED_DIM)

            def cbody(c, carry):
                for l0 in range(BW // LANES):
                    col = half[l0] + c
                    v = plsc.load_gather(gbuf.at[sp], [row_ids[l0], col])
                    sbuf[sp, c, pl.ds(l0 * LANES, LANES)] = v * SCALE
                return carry

            lax.fori_loop(0, EMBED_DIM, cbody, 0, unroll=2)

        for k in range(LOOKAHEAD):
            fire_gather(k, k)

        def group_body(g0, carry):
            for sp in range(NBUF):
                g = g0 * NBUF + sp
                pf = g + LOOKAHEAD
                sp_pf = (sp + LOOKAHEAD) % NBUF

                @pl.when(pf < n_seq)
                def _():
                    @pl.when(pf >= NBUF)
                    def _():
                        wait_store(pf - NBUF, sp_pf)

                    fire_gather(pf, sp_pf)

                wait_gather(sp)
                compact(sp)
                store_chunk(g, sp)
            return carry

        lax.fori_loop(0, n_seq // NBUF, group_body, 0)

        for sp in range(NBUF):
            wait_store(n_seq - NBUF + sp, sp)

    return emb


def kernel(x, table):
    b, s = x.shape
    xt = jnp.swapaxes(x, 0, 1).astype(jnp.int32)
    pairs = table.reshape(-1, 128)
    out = _emb_kernel(s, b)(xt, pairs)
    return jnp.transpose(out, (2, 0, 1))


# speedup vs baseline: 1.6116x; 1.6116x over previous
import functools

import jax
import jax.numpy as jnp
from jax import lax
from jax.experimental import pallas as pl
from jax.experimental.pallas import tpu as pltpu
from jax.experimental.pallas import tpu_sc as plsc

EMBED_DIM = 64
SCALE = 8.0
NUM_WORKERS = 32
BW = 128
NBUF = 4
LOOKAHEAD = 2
LANES = 16


def _emb_kernel(n_seq, n_batch):
    assert n_batch == NUM_WORKERS * BW
    assert n_seq % NBUF == 0 and n_seq >= 2 * NBUF
    mesh = plsc.VectorSubcoreMesh(core_axis_name="c", subcore_axis_name="s")

    @functools.partial(
        pl.kernel,
        mesh=mesh,
        out_type=jax.ShapeDtypeStruct((n_seq, EMBED_DIM, n_batch), jnp.float32),
        scratch_types=[
            pltpu.VMEM((NBUF, BW), jnp.int32),
            pltpu.VMEM((NBUF, BW), jnp.int32),
            pltpu.VMEM((NBUF, BW, 128), jnp.float32),
            pltpu.VMEM((NBUF, EMBED_DIM, BW), jnp.float32),
            pltpu.SemaphoreType.DMA((NBUF,)),
            pltpu.SemaphoreType.DMA((NBUF,)),
        ],
        compiler_params=pltpu.CompilerParams(needs_layout_passes=False),
    )
    def emb(xt_hbm, pairs_hbm, out_hbm, idxr, idxh, gbuf, sbuf, sem_g, sem_o):
        wid = lax.axis_index("s") * 2 + lax.axis_index("c")
        b0 = wid * BW

        lane = lax.iota(jnp.int32, LANES)
        rot = [(lane + cc) & (LANES - 1) for cc in range(LANES)]

        def fire_gather(g, sp):
            pltpu.sync_copy(xt_hbm.at[g, pl.ds(b0, BW)], idxr.at[sp])
            for k in range(BW // LANES):
                sl = pl.ds(k * LANES, LANES)
                idxh[sp, sl] = lax.shift_right_logical(idxr[sp, sl], 1)
            pltpu.async_copy(
                pairs_hbm.at[idxh.at[sp]], gbuf.at[sp], sem_g.at[sp]
            )

        def wait_gather(sp):
            pltpu.make_async_copy(
                pairs_hbm.at[idxh.at[sp]], gbuf.at[sp], sem_g.at[sp]
            ).wait()

        def store_chunk(g, sp):
            pltpu.async_copy(
                sbuf.at[sp], out_hbm.at[g, :, pl.ds(b0, BW)], sem_o.at[sp]
            )

        def wait_store(g, sp):
            pltpu.make_async_copy(
                sbuf.at[sp], out_hbm.at[g, :, pl.ds(b0, BW)], sem_o.at[sp]
            ).wait()

        def compact(sp):
            def do_group(l0, carry):
                base = l0 * LANES
                iv = idxr[sp, pl.ds(base, LANES)]
                half = (iv & 1) * EMBED_DIM
                rowv = lane + base
                for d in range(EMBED_DIM // LANES):
                    for cc in range(LANES):
                        cdv = rot[cc] + (d * LANES)
                        colv = cdv + half
                        v = plsc.load_gather(gbuf.at[sp], [rowv, colv])
                        plsc.store_scatter(sbuf.at[sp], [cdv, rowv], v * SCALE)
                return carry

            lax.fori_loop(0, BW // LANES, do_group, 0)

        for k in range(LOOKAHEAD):
            fire_gather(k, k)

        def group_body(g0, carry):
            for sp in range(NBUF):
                g = g0 * NBUF + sp
                pf = g + LOOKAHEAD
                sp_pf = (sp + LOOKAHEAD) % NBUF

                @pl.when(pf < n_seq)
                def _():
                    @pl.when(pf >= NBUF)
                    def _():
                        wait_store(pf - NBUF, sp_pf)

                    fire_gather(pf, sp_pf)

                wait_gather(sp)
                compact(sp)
                store_chunk(g, sp)
            return carry

        lax.fori_loop(0, n_seq // NBUF, group_body, 0)

        for sp in range(NBUF):
            wait_store(n_seq - NBUF + sp, sp)

    return emb


def kernel(x, table):
    b, s = x.shape
    xt = jnp.swapaxes(x, 0, 1).astype(jnp.int32)
    pairs = table.reshape(-1, 128)
    out = _emb_kernel(s, b)(xt, pairs)
    return jnp.transpose(out, (2, 0, 1))


# speedup vs baseline: 2.2829x; 1.4166x over previous
import functools

import jax
import jax.numpy as jnp
from jax import lax
from jax.experimental import pallas as pl
from jax.experimental.pallas import tpu as pltpu
from jax.experimental.pallas import tpu_sc as plsc

EMBED_DIM = 64
SCALE = 8.0
NUM_WORKERS = 32
BW = 128
NBUF = 4
LOOKAHEAD = 2
LANES = 16


def _emb_kernel(n_seq, n_batch):
    assert n_batch == NUM_WORKERS * BW
    assert n_seq % NBUF == 0 and n_seq >= 2 * NBUF
    mesh = plsc.VectorSubcoreMesh(core_axis_name="c", subcore_axis_name="s")

    @functools.partial(
        pl.kernel,
        mesh=mesh,
        out_type=jax.ShapeDtypeStruct((n_seq, EMBED_DIM, n_batch), jnp.float32),
        scratch_types=[
            pltpu.VMEM((NBUF, BW), jnp.int32),
            pltpu.VMEM((NBUF, BW), jnp.int32),
            pltpu.VMEM((NBUF, BW, 128), jnp.float32),
            pltpu.VMEM((NBUF, EMBED_DIM, BW), jnp.float32),
            pltpu.SemaphoreType.DMA((NBUF,)),
            pltpu.SemaphoreType.DMA((NBUF,)),
        ],
        compiler_params=pltpu.CompilerParams(needs_layout_passes=False),
    )
    def emb(xt_hbm, pairs_hbm, out_hbm, idxr, idxh, gbuf, sbuf, sem_g, sem_o):
        wid = lax.axis_index("s") * 2 + lax.axis_index("c")
        b0 = wid * BW

        lane = lax.iota(jnp.int32, LANES)
        rot = [(lane + cc) & (LANES - 1) for cc in range(LANES)]

        def fire_gather(g, sp):
            pltpu.sync_copy(xt_hbm.at[g, pl.ds(b0, BW)], idxr.at[sp])
            for k in range(BW // LANES):
                sl = pl.ds(k * LANES, LANES)
                idxh[sp, sl] = lax.shift_right_logical(idxr[sp, sl], 1)
            pltpu.async_copy(
                pairs_hbm.at[idxh.at[sp]], gbuf.at[sp], sem_g.at[sp]
            )

        def wait_gather(sp):
            pltpu.make_async_copy(
                pairs_hbm.at[idxh.at[sp]], gbuf.at[sp], sem_g.at[sp]
            ).wait()

        def store_chunk(g, sp):
            pltpu.async_copy(
                sbuf.at[sp], out_hbm.at[g, :, pl.ds(b0, BW)], sem_o.at[sp]
            )

        def wait_store(g, sp):
            pltpu.make_async_copy(
                sbuf.at[sp], out_hbm.at[g, :, pl.ds(b0, BW)], sem_o.at[sp]
            ).wait()

        def compact(sp):
            n_quanta = (BW // LANES) * (EMBED_DIM // LANES) * LANES

            @plsc.parallel_loop(0, n_quanta, unroll=4)
            def _(q):
                l0 = q >> 6
                dd = (q >> 4) & 3
                cc = q & (LANES - 1)
                iv = idxr[sp, pl.ds(l0 * LANES, LANES)]
                half = (iv & 1) * EMBED_DIM
                rowv = lane + l0 * LANES
                cdv = ((lane + cc) & (LANES - 1)) + dd * LANES
                colv = cdv + half
                v = plsc.load_gather(gbuf.at[sp], [rowv, colv])
                plsc.store_scatter(sbuf.at[sp], [cdv, rowv], v * SCALE)

        for k in range(LOOKAHEAD):
            fire_gather(k, k)

        def group_body(g0, carry):
            for sp in range(NBUF):
                g = g0 * NBUF + sp
                pf = g + LOOKAHEAD
                sp_pf = (sp + LOOKAHEAD) % NBUF

                @pl.when(pf < n_seq)
                def _():
                    @pl.when(pf >= NBUF)
                    def _():
                        wait_store(pf - NBUF, sp_pf)

                    fire_gather(pf, sp_pf)

                wait_gather(sp)
                compact(sp)
                store_chunk(g, sp)
            return carry

        lax.fori_loop(0, n_seq // NBUF, group_body, 0)

        for sp in range(NBUF):
            wait_store(n_seq - NBUF + sp, sp)

    return emb


def kernel(x, table):
    b, s = x.shape
    xt = jnp.swapaxes(x, 0, 1).astype(jnp.int32)
    pairs = table.reshape(-1, 128)
    out = _emb_kernel(s, b)(xt, pairs)
    return jnp.transpose(out, (2, 0, 1))


# speedup vs baseline: 2.5169x; 1.1025x over previous
import functools

import jax
import jax.numpy as jnp
from jax import lax
from jax.experimental import pallas as pl
from jax.experimental.pallas import tpu as pltpu
from jax.experimental.pallas import tpu_sc as plsc

EMBED_DIM = 64
SCALE = 8.0
NUM_WORKERS = 32
BW = 128
NBUF = 4
LOOKAHEAD = 2
LANES = 16


def _emb_kernel(n_seq, n_batch):
    assert n_batch == NUM_WORKERS * BW
    assert n_seq % NBUF == 0 and n_seq >= 2 * NBUF
    mesh = plsc.VectorSubcoreMesh(core_axis_name="c", subcore_axis_name="s")

    @functools.partial(
        pl.kernel,
        mesh=mesh,
        out_type=jax.ShapeDtypeStruct((n_seq, EMBED_DIM, n_batch), jnp.float32),
        scratch_types=[
            pltpu.VMEM((NBUF, BW), jnp.int32),
            pltpu.VMEM((NBUF, BW), jnp.int32),
            pltpu.VMEM((NBUF, BW, 128), jnp.float32),
            pltpu.VMEM((NBUF, EMBED_DIM, BW), jnp.float32),
            pltpu.SemaphoreType.DMA((NBUF,)),
            pltpu.SemaphoreType.DMA((NBUF,)),
        ],
        compiler_params=pltpu.CompilerParams(needs_layout_passes=False),
    )
    def emb(xt_hbm, pairs_hbm, out_hbm, idxr, idxh, gbuf, sbuf, sem_g, sem_o):
        wid = lax.axis_index("s") * 2 + lax.axis_index("c")
        b0 = wid * BW

        lane = lax.iota(jnp.int32, LANES)
        rot = [(lane + cc) & (LANES - 1) for cc in range(LANES)]

        def fire_gather(g, sp):
            pltpu.sync_copy(xt_hbm.at[g, pl.ds(b0, BW)], idxr.at[sp])
            for k in range(BW // LANES):
                sl = pl.ds(k * LANES, LANES)
                idxh[sp, sl] = lax.shift_right_logical(idxr[sp, sl], 1)
            pltpu.async_copy(
                pairs_hbm.at[idxh.at[sp]], gbuf.at[sp], sem_g.at[sp]
            )

        def wait_gather(sp):
            pltpu.make_async_copy(
                pairs_hbm.at[idxh.at[sp]], gbuf.at[sp], sem_g.at[sp]
            ).wait()

        def store_chunk(g, sp):
            pltpu.async_copy(
                sbuf.at[sp], out_hbm.at[g, :, pl.ds(b0, BW)], sem_o.at[sp]
            )

        def wait_store(g, sp):
            pltpu.make_async_copy(
                sbuf.at[sp], out_hbm.at[g, :, pl.ds(b0, BW)], sem_o.at[sp]
            ).wait()

        def compact(sp):
            n_quanta = (BW // LANES) * (EMBED_DIM // LANES) * LANES

            @plsc.parallel_loop(0, n_quanta, unroll=4)
            def _(q):
                l0 = q >> 6
                dd = (q >> 4) & 3
                cc = q & (LANES - 1)
                iv = idxr[sp, pl.ds(l0 * LANES, LANES)]
                half = (iv & 1) * EMBED_DIM
                rowv = lane + l0 * LANES
                cdv = ((lane + cc) & (LANES - 1)) + dd * LANES
                colv = cdv + half
                v = plsc.load_gather(gbuf.at[sp], [rowv, colv])
                plsc.store_scatter(sbuf.at[sp], [cdv, rowv], v * SCALE)

        for k in range(LOOKAHEAD):
            fire_gather(k, k)

        def group_body(g0, carry):
            for sp in range(NBUF):
                g = g0 * NBUF + sp
                pf = g + LOOKAHEAD
                sp_pf = (sp + LOOKAHEAD) % NBUF

                @pl.when(pf < n_seq)
                def _():
                    @pl.when(pf >= NBUF)
                    def _():
                        wait_store(pf - NBUF, sp_pf)

                    fire_gather(pf, sp_pf)

                wait_gather(sp)
                compact(sp)
                store_chunk(g, sp)
            return carry

        lax.fori_loop(0, n_seq // NBUF, group_body, 0)

        for sp in range(NBUF):
            wait_store(n_seq - NBUF + sp, sp)

    return emb


RCHUNK = 320


def _repack_kernel(vocab):
    n_chunks = vocab // RCHUNK
    mesh = plsc.VectorSubcoreMesh(core_axis_name="c", subcore_axis_name="s")
    n_iter = (n_chunks + NUM_WORKERS - 1) // NUM_WORKERS

    @functools.partial(
        pl.kernel,
        mesh=mesh,
        out_type=jax.ShapeDtypeStruct((vocab // 2, 128), jnp.float32),
        scratch_types=[
            pltpu.VMEM((2, RCHUNK, EMBED_DIM), jnp.float32),
            pltpu.VMEM((2, RCHUNK // 2, 128), jnp.float32),
            pltpu.SemaphoreType.DMA((2,)),
            pltpu.SemaphoreType.DMA((2,)),
        ],
        compiler_params=pltpu.CompilerParams(needs_layout_passes=False),
    )
    def rep(tbl_hbm, pairs_hbm, tin, tout, sem_i, sem_o):
        wid = lax.axis_index("s") * 2 + lax.axis_index("c")

        def _in_off(ci):
            return pl.multiple_of((wid + NUM_WORKERS * ci) * RCHUNK, RCHUNK)

        def _out_off(ci):
            return pl.multiple_of(
                (wid + NUM_WORKERS * ci) * (RCHUNK // 2), RCHUNK // 2
            )

        def fire_in(ci, sl):
            pltpu.async_copy(
                tbl_hbm.at[pl.ds(_in_off(ci), RCHUNK)], tin.at[sl], sem_i.at[sl]
            )

        def wait_in(ci, sl):
            pltpu.make_async_copy(
                tbl_hbm.at[pl.ds(_in_off(ci), RCHUNK)], tin.at[sl], sem_i.at[sl]
            ).wait()

        def fire_out(ci, sl):
            pltpu.async_copy(
                tout.at[sl], pairs_hbm.at[pl.ds(_out_off(ci), RCHUNK // 2)],
                sem_o.at[sl],
            )

        def wait_out(ci, sl):
            pltpu.make_async_copy(
                tout.at[sl], pairs_hbm.at[pl.ds(_out_off(ci), RCHUNK // 2)],
                sem_o.at[sl],
            ).wait()

        @pl.when(wid < n_chunks)
        def _():
            fire_in(0, 0)

        def one_step(ci, sl):
            gid = wid + NUM_WORKERS * ci

            @pl.when(gid < n_chunks)
            def _():
                @pl.when(gid + NUM_WORKERS < n_chunks)
                def _():
                    @pl.when(gid >= NUM_WORKERS)
                    def _():
                        wait_out(ci - 1, 1 - sl)

                    fire_in(ci + 1, 1 - sl)

                wait_in(ci, sl)

                @plsc.parallel_loop(0, RCHUNK // 2, unroll=4)
                def _(k):
                    for half in range(2):
                        for d in range(EMBED_DIM // LANES):
                            tout[sl, k, pl.ds(half * EMBED_DIM + d * LANES, LANES)] = (
                                tin[sl, 2 * k + half, pl.ds(d * LANES, LANES)]
                            )

                fire_out(ci, sl)

        def body(c2, carry):
            for sls in range(2):
                one_step(c2 * 2 + sls, sls)
            return carry

        lax.fori_loop(0, (n_iter + 1) // 2, body, 0)

        for back in range(4, 0, -1):
            ci = (n_iter + 1) // 2 * 2 - back
            if ci < 0:
                continue
            gid = wid + NUM_WORKERS * ci

            @pl.when((gid < n_chunks) & (gid + 2 * NUM_WORKERS >= n_chunks))
            def _():
                wait_out(ci, ci % 2)

    return rep


def kernel(x, table):
    b, s = x.shape
    v, _ = table.shape
    xt = jnp.swapaxes(x, 0, 1).astype(jnp.int32)
    pairs = _repack_kernel(v)(table)
    out = _emb_kernel(s, b)(xt, pairs)
    return jnp.transpose(out, (2, 0, 1))


# speedup vs baseline: 3.5819x; 1.4232x over previous
import functools

import jax
import jax.numpy as jnp
from jax import lax
from jax.experimental import pallas as pl
from jax.experimental.pallas import tpu as pltpu
from jax.experimental.pallas import tpu_sc as plsc

EMBED_DIM = 64
SCALE = 8.0
NUM_WORKERS = 32
BW = 128
NBUF = 4
LOOKAHEAD = 2
LANES = 16


def _emb_kernel(n_seq, n_batch):
    assert n_batch == NUM_WORKERS * BW
    assert n_seq % NBUF == 0 and n_seq >= 2 * NBUF
    mesh = plsc.VectorSubcoreMesh(core_axis_name="c", subcore_axis_name="s")

    @functools.partial(
        pl.kernel,
        mesh=mesh,
        out_type=jax.ShapeDtypeStruct((n_seq, EMBED_DIM, n_batch), jnp.float32),
        scratch_types=[
            pltpu.VMEM((NBUF, BW), jnp.int32),
            pltpu.VMEM((NBUF, BW), jnp.int32),
            pltpu.VMEM((NBUF, BW, 128), jnp.float32),
            pltpu.VMEM((NBUF, EMBED_DIM, BW), jnp.float32),
            pltpu.SemaphoreType.DMA((NBUF,)),
            pltpu.SemaphoreType.DMA((NBUF,)),
        ],
        compiler_params=pltpu.CompilerParams(needs_layout_passes=False),
    )
    def emb(xt_hbm, pairs_hbm, out_hbm, idxr, idxh, gbuf, sbuf, sem_g, sem_o):
        wid = lax.axis_index("s") * 2 + lax.axis_index("c")
        b0 = wid * BW

        lane = lax.iota(jnp.int32, LANES)
        rot = [(lane + cc) & (LANES - 1) for cc in range(LANES)]

        def fire_gather(g, sp):
            pltpu.sync_copy(xt_hbm.at[g, pl.ds(b0, BW)], idxr.at[sp])
            for k in range(BW // LANES):
                sl = pl.ds(k * LANES, LANES)
                idxh[sp, sl] = lax.shift_right_logical(idxr[sp, sl], 1)
            pltpu.async_copy(
                pairs_hbm.at[idxh.at[sp]], gbuf.at[sp], sem_g.at[sp]
            )

        def wait_gather(sp):
            pltpu.make_async_copy(
                pairs_hbm.at[idxh.at[sp]], gbuf.at[sp], sem_g.at[sp]
            ).wait()

        def store_chunk(g, sp):
            pltpu.async_copy(
                sbuf.at[sp], out_hbm.at[g, :, pl.ds(b0, BW)], sem_o.at[sp]
            )

        def wait_store(g, sp):
            pltpu.make_async_copy(
                sbuf.at[sp], out_hbm.at[g, :, pl.ds(b0, BW)], sem_o.at[sp]
            ).wait()

        def compact(sp):
            n_quanta = (BW // LANES) * (EMBED_DIM // LANES) * LANES

            @plsc.parallel_loop(0, n_quanta, unroll=4)
            def _(q):
                l0 = q >> 6
                dd = (q >> 4) & 3
                cc = q & (LANES - 1)
                iv = idxr[sp, pl.ds(l0 * LANES, LANES)]
                half = (iv & 1) * EMBED_DIM
                rowv = lane + l0 * LANES
                cdv = ((lane + cc) & (LANES - 1)) + dd * LANES
                colv = cdv + half
                v = plsc.load_gather(gbuf.at[sp], [rowv, colv])
                plsc.store_scatter(sbuf.at[sp], [cdv, rowv], v * SCALE)

        for k in range(LOOKAHEAD):
            fire_gather(k, k)

        def group_body(g0, carry):
            for sp in range(NBUF):
                g = g0 * NBUF + sp
                pf = g + LOOKAHEAD
                sp_pf = (sp + LOOKAHEAD) % NBUF

                @pl.when(pf < n_seq)
                def _():
                    @pl.when(pf >= NBUF)
                    def _():
                        wait_store(pf - NBUF, sp_pf)

                    fire_gather(pf, sp_pf)

                wait_gather(sp)
                compact(sp)
                store_chunk(g, sp)
            return carry

        lax.fori_loop(0, n_seq // NBUF, group_body, 0)

        for sp in range(NBUF):
            wait_store(n_seq - NBUF + sp, sp)

    return emb


TCH = 128
RNB = 4


def _repack_kernel(vocab):
    n_chunks = vocab // TCH
    mesh = plsc.VectorSubcoreMesh(core_axis_name="c", subcore_axis_name="s")
    n_iter = (n_chunks + NUM_WORKERS - 1) // NUM_WORKERS

    @functools.partial(
        pl.kernel,
        mesh=mesh,
        out_type=jax.ShapeDtypeStruct((vocab // 2, 128), jnp.float32),
        scratch_types=[
            pltpu.VMEM((RNB, EMBED_DIM, TCH), jnp.float32),
            pltpu.VMEM((RNB, TCH // 2, 128), jnp.float32),
            pltpu.VMEM((128, LANES), jnp.int32),
            pltpu.VMEM((128, LANES), jnp.int32),
            pltpu.SemaphoreType.DMA((RNB,)),
            pltpu.SemaphoreType.DMA((RNB,)),
        ],
        compiler_params=pltpu.CompilerParams(needs_layout_passes=False),
    )
    def rep(tt_hbm, tail_hbm, pairs_hbm, tin, tout, prow, pbit, sem_i, sem_o):
        wid = lax.axis_index("s") * 2 + lax.axis_index("c")
        lane = lax.iota(jnp.int32, LANES)

        @plsc.parallel_loop(0, 128, unroll=4)
        def _(p):
            dj = p >> 4
            cc = p & (LANES - 1)
            jp = ((lane + cc) & (LANES - 1)) + dj * LANES
            prow[p, pl.ds(0, LANES)] = jp & (EMBED_DIM - 1)
            pbit[p, pl.ds(0, LANES)] = (jp >= EMBED_DIM).astype(jnp.int32)

        k2 = [lane * 2 + (kg * LANES * 2) for kg in range(TCH // 2 // LANES)]
        krow = [lane + (kg * LANES) for kg in range(TCH // 2 // LANES)]

        def _in_off(ci):
            return pl.multiple_of((wid + NUM_WORKERS * ci) * TCH, TCH)

        def _out_off(ci):
            return pl.multiple_of((wid + NUM_WORKERS * ci) * (TCH // 2), TCH // 2)

        def fire_in(ci, sl):
            pltpu.async_copy(
                tt_hbm.at[:, pl.ds(_in_off(ci), TCH)], tin.at[sl], sem_i.at[sl]
            )

        def wait_in(ci, sl):
            pltpu.make_async_copy(
                tt_hbm.at[:, pl.ds(_in_off(ci), TCH)], tin.at[sl], sem_i.at[sl]
            ).wait()

        def fire_out(ci, sl):
            pltpu.async_copy(
                tout.at[sl], pairs_hbm.at[pl.ds(_out_off(ci), TCH // 2)],
                sem_o.at[sl],
            )

        def wait_out(ci, sl):
            pltpu.make_async_copy(
                tout.at[sl], pairs_hbm.at[pl.ds(_out_off(ci), TCH // 2)],
                sem_o.at[sl],
            ).wait()

        @pl.when(wid == 0)
        def _():
            tail_rows = (vocab - n_chunks * TCH) // 2
            pltpu.sync_copy(tail_hbm, tout.at[0].at[pl.ds(0, tail_rows)])
            pltpu.sync_copy(
                tout.at[0].at[pl.ds(0, tail_rows)],
                pairs_hbm.at[pl.ds(n_chunks * TCH // 2, tail_rows)],
            )

        for k in range(2):
            @pl.when(wid + NUM_WORKERS * k < n_chunks)
            def _():
                fire_in(k, k)

        def one_step(ci, sl):
            gid = wid + NUM_WORKERS * ci

            @pl.when(gid < n_chunks)
            def _():
                pf = ci + 2
                sl_pf = (sl + 2) % RNB

                @pl.when(gid + 2 * NUM_WORKERS < n_chunks)
                def _():
                    @pl.when(gid >= 2 * NUM_WORKERS)
                    def _():
                        wait_out(pf - RNB, sl_pf)

                    fire_in(pf, sl_pf)

                wait_in(ci, sl)

                for kg in range(TCH // 2 // LANES):
                    @plsc.parallel_loop(0, 128, unroll=4)
                    def _(p):
                        rowv = prow[p, pl.ds(0, LANES)]
                        bitv = pbit[p, pl.ds(0, LANES)]
                        colv = k2[kg] + bitv
                        v = plsc.load_gather(tin.at[sl], [rowv, colv])
                        plsc.store_scatter(
                            tout.at[sl],
                            [krow[kg], rowv + bitv * EMBED_DIM],
                            v,
                        )

                fire_out(ci, sl)

        def body(c2, carry):
            for sls in range(RNB):
                one_step(c2 * RNB + sls, sls)
            return carry

        lax.fori_loop(0, (n_iter + RNB - 1) // RNB, body, 0)

        tot = (n_iter + RNB - 1) // RNB * RNB
        for back in range(2 * RNB, 0, -1):
            ci = tot - back
            if ci < 0:
                continue
            gid = wid + NUM_WORKERS * ci

            @pl.when((gid < n_chunks) & (gid + 4 * NUM_WORKERS >= n_chunks))
            def _():
                wait_out(ci, ci % RNB)

    return rep


def kernel(x, table):
    b, s = x.shape
    v, _ = table.shape
    xt = jnp.swapaxes(x, 0, 1).astype(jnp.int32)
    tt = jnp.swapaxes(table, 0, 1)
    n_full = v // TCH * TCH
    tail = table[n_full:].reshape(-1, 128)
    pairs = _repack_kernel(v)(tt, tail)
    out = _emb_kernel(s, b)(xt, pairs)
    return jnp.transpose(out, (2, 0, 1))
